# baseline (speedup 1.0000x reference)
import functools

import jax
import jax.numpy as jnp
from jax import lax
from jax.experimental import pallas as pl
from jax.experimental.pallas import tpu as pltpu
from jax.experimental.pallas import tpu_sc as plsc

_N = 10000
_E = 320000
_CH = 128
_NC = 2
_NS = 16
_NW = _NC * _NS
_EPW = _E // _NW
_K = 128
_EPWP = 10240
_NCHUNK = _EPWP // _K
_NP = 10240
_RPT = _NP // _NS
_NH = 640
_REP = 64


def _sc_gs_body(tab_hbm, row_hbm, col_hbm, zeros_hbm, out_hbm,
                row_v, col_v, rows_v, acc, sem):
    rpt = acc.shape[0] // _NS
    core = lax.axis_index("c")
    sid = lax.axis_index("s")
    wid = sid * _NC + core
    pltpu.sync_copy(row_hbm.at[wid], row_v)
    pltpu.sync_copy(col_hbm.at[wid], col_v)
    pltpu.sync_copy(zeros_hbm.at[pl.ds(sid * rpt, rpt)],
                    acc.at[pl.ds(sid * rpt, rpt)])
    plsc.subcore_barrier()

    def chunk(j, carry):
        pltpu.async_copy(tab_hbm.at[row_v.at[j]], rows_v, sem).wait()
        pltpu.sync_copy(rows_v, acc.at[col_v.at[j]], add=True)
        return carry

    lax.fori_loop(0, _NCHUNK, chunk, 0)
    plsc.subcore_barrier()
    pltpu.sync_copy(acc.at[pl.ds(sid * rpt, rpt)],
                    out_hbm.at[core, pl.ds(sid * rpt, rpt)])


@functools.partial(jax.jit, static_argnums=(4,))
def _sc_gather_scatter(tab, row3, col3, zeros, n_acc):
    mesh = plsc.VectorSubcoreMesh(core_axis_name="c", subcore_axis_name="s")
    return pl.kernel(
        _sc_gs_body,
        out_type=jax.ShapeDtypeStruct((_NC, n_acc, _CH), jnp.float32),
        mesh=mesh,
        scratch_types=[
            pltpu.VMEM((_NCHUNK, _K), jnp.int32),
            pltpu.VMEM((_NCHUNK, _K), jnp.int32),
            pltpu.VMEM((_K, _CH), jnp.float32),
            pltpu.VMEM_SHARED((n_acc, _CH), jnp.float32),
            pltpu.SemaphoreType.DMA,
        ],
    )(tab, row3, col3, zeros)


def _tc_xw_body(state_ref, w_ref, xw_ref):
    xw_ref[...] = jnp.dot(state_ref[...], w_ref[...],
                          preferred_element_type=jnp.float32)


@jax.jit
def _tc_xw(state, conv_W):
    grid = 25
    rb = _N // grid
    return pl.pallas_call(
        _tc_xw_body,
        grid=(grid,),
        in_specs=[
            pl.BlockSpec((rb, _CH), lambda i: (i, 0)),
            pl.BlockSpec((_CH, _CH), lambda i: (0, 0)),
        ],
        out_specs=pl.BlockSpec((rb, _CH), lambda i: (i, 0)),
        out_shape=jax.ShapeDtypeStruct((_N, _CH), jnp.float32),
    )(state, conv_W)


def _tc_xs_body(xw_ref, h0_ref, h1_ref, xs_ref, dis_ref):
    deg = h0_ref[...] + h1_ref[...] + 1.0
    dis = lax.rsqrt(deg)
    xs_ref[...] = dis * xw_ref[...]
    dis_ref[...] = dis


@jax.jit
def _tc_xs(xw, h0, h1):
    grid = 25
    rb = _N // grid
    return pl.pallas_call(
        _tc_xs_body,
        grid=(grid,),
        in_specs=[
            pl.BlockSpec((rb, _CH), lambda i: (i, 0)),
            pl.BlockSpec((rb, 1), lambda i: (i, 0)),
            pl.BlockSpec((rb, 1), lambda i: (i, 0)),
        ],
        out_specs=[
            pl.BlockSpec((rb, _CH), lambda i: (i, 0)),
            pl.BlockSpec((rb, 1), lambda i: (i, 0)),
        ],
        out_shape=[
            jax.ShapeDtypeStruct((_N, _CH), jnp.float32),
            jax.ShapeDtypeStruct((_N, 1), jnp.float32),
        ],
    )(xw, h0, h1)


def _tc_final_body(a0_ref, a1_ref, xs_ref, dis_ref, state_ref, cb_ref, s_ref,
                   w1_ref, b1_ref, w2_ref, b2_ref, w3_ref, b3_ref, out_ref):
    conv = dis_ref[...] * (a0_ref[...] + a1_ref[...] + xs_ref[...]) + cb_ref[...]
    h = jnp.maximum(conv, 0.0) + state_ref[...]
    g = jnp.dot(s_ref[...], h, preferred_element_type=jnp.float32)
    z = jnp.maximum(jnp.dot(g, w1_ref[...], preferred_element_type=jnp.float32)
                    + b1_ref[...], 0.0)
    z = jnp.maximum(jnp.dot(z, w2_ref[...], preferred_element_type=jnp.float32)
                    + b2_ref[...], 0.0)
    y = jnp.dot(z, w3_ref[...], preferred_element_type=jnp.float32) + b3_ref[...]
    out_ref[...] = jnp.broadcast_to(y[None], out_ref.shape)


@jax.jit
def _tc_final(a0, a1, xs, dis, state, conv_b, sel,
              lin1_W, lin1_b, lin2_W, lin2_b, lin3_W, lin3_b):
    grid = 25
    rb = _N // grid
    gb = rb // 16
    full = lambda shape: pl.BlockSpec(shape, lambda i: tuple(0 for _ in shape))
    return pl.pallas_call(
        _tc_final_body,
        grid=(grid,),
        in_specs=[
            pl.BlockSpec((rb, _CH), lambda i: (i, 0)),
            pl.BlockSpec((rb, _CH), lambda i: (i, 0)),
            pl.BlockSpec((rb, _CH), lambda i: (i, 0)),
            pl.BlockSpec((rb, 1), lambda i: (i, 0)),
            pl.BlockSpec((rb, _CH), lambda i: (i, 0)),
            full((1, _CH)),
            full((gb, rb)),
            full((_CH, 64)),
            full((1, 64)),
            full((64, 64)),
            full((1, 64)),
            full((64, 1)),
            full((1, 1)),
        ],
        out_specs=pl.BlockSpec((1, gb, _CH), lambda i: (i, 0, 0)),
        out_shape=jax.ShapeDtypeStruct((grid, gb, _CH), jnp.float32),
    )(a0, a1, xs, dis, state, conv_b, sel,
      lin1_W, lin1_b, lin2_W, lin2_b, lin3_W, lin3_b)


def kernel(state, edge_index, conv_W, conv_b, lin1_W, lin1_b, lin2_W, lin2_b,
           lin3_W, lin3_b):
    npad = _NW * _EPWP - _E
    arp = jnp.arange(npad, dtype=jnp.int32)
    row = jnp.concatenate([edge_index[0].astype(jnp.int32), (arp * 7) % _N])
    col = edge_index[1].astype(jnp.int32)
    row32 = row.reshape(_NW, _NCHUNK, _K)
    col32 = jnp.concatenate([col, _N + (arp % (_NP - _N))]).reshape(
        _NW, _NCHUNK, _K)
    zeros = jnp.zeros((_NP, _CH), jnp.float32)

    ar = jnp.arange(_E, dtype=jnp.int32)
    hrow = jnp.concatenate([(col & 15) * _REP + (ar % _REP),
                            arp % (16 * _REP)]).reshape(_NW, _NCHUNK, _K)
    hcol = jnp.concatenate([col >> 4,
                            625 + (arp % (_NH - 625))]).reshape(
        _NW, _NCHUNK, _K)
    m = jnp.arange(16 * _REP, dtype=jnp.int32)
    onehot = jnp.zeros((16 * _REP, _CH), jnp.float32).at[m, 8 * (m // _REP)].set(1.0)
    zeros_h = jnp.zeros((_NH, _CH), jnp.float32)
    hist = _sc_gather_scatter(onehot, hrow, hcol, zeros_h, _NH)
    h0 = hist[0].reshape(_NH, 16, 8)[:, :, 0].reshape(-1, 1)[:_N]
    h1 = hist[1].reshape(_NH, 16, 8)[:, :, 0].reshape(-1, 1)[:_N]

    xw = _tc_xw(state, conv_W)
    xs, dis = _tc_xs(xw, h0, h1)
    acc = _sc_gather_scatter(xs, row32, col32, zeros, _NP)

    sel = (jnp.arange(25)[:, None] == (jnp.arange(400) // 16)[None, :]
           ).astype(jnp.float32)
    out3d = _tc_final(acc[0, :_N], acc[1, :_N], xs, dis, state,
                      conv_b.reshape(1, _CH), sel, lin1_W,
                      lin1_b.reshape(1, 64), lin2_W, lin2_b.reshape(1, 64),
                      lin3_W, lin3_b.reshape(1, 1))
    return out3d.reshape(_N // 16, _CH)[:, 0]


# speedup vs baseline: 1.1048x; 1.1048x over previous
import functools

import jax
import jax.numpy as jnp
from jax import lax
from jax.experimental import pallas as pl
from jax.experimental.pallas import tpu as pltpu
from jax.experimental.pallas import tpu_sc as plsc

_N = 10000
_E = 320000
_CH = 128
_NC = 2
_NS = 16
_NW = _NC * _NS
_EPW = _E // _NW
_K = 128
_EPWP = 10240
_NCHUNK = _EPWP // _K
_CB = 8
_NP = 10240
_RPT = _NP // _NS
_NH = 640
_REP = 64


def _sc_gs_body(tab_hbm, row_hbm, col_hbm, zeros_hbm, out_hbm,
                row_v, col_b, rows_a, rows_b, acc, gsa, gsb, ssa, ssb):
    rpt = acc.shape[0] // _NS
    core = lax.axis_index("c")
    sid = lax.axis_index("s")
    wid = sid * _NC + core
    pltpu.sync_copy(row_hbm.at[wid], row_v)
    pltpu.sync_copy(zeros_hbm.at[pl.ds(sid * rpt, rpt)],
                    acc.at[pl.ds(sid * rpt, rpt)])
    plsc.subcore_barrier()

    def block(q, carry):
        pltpu.sync_copy(col_hbm.at[wid, pl.ds(_CB * q, _CB)], col_b)
        for pp in range(_CB // 2):
            c0 = _CB * q + 2 * pp
            dg0 = pltpu.async_copy(tab_hbm.at[row_v.at[c0]], rows_a, gsa)
            dg1 = pltpu.async_copy(tab_hbm.at[row_v.at[c0 + 1]], rows_b, gsb)
            dg0.wait()
            dg1.wait()
            ds0 = pltpu.async_copy(rows_a, acc.at[col_b.at[2 * pp]], ssa,
                                   add=True)
            ds1 = pltpu.async_copy(rows_b, acc.at[col_b.at[2 * pp + 1]], ssb,
                                   add=True)
            ds0.wait()
            ds1.wait()
        return carry

    lax.fori_loop(0, _NCHUNK // _CB, block, 0)
    plsc.subcore_barrier()
    pltpu.sync_copy(acc.at[pl.ds(sid * rpt, rpt)],
                    out_hbm.at[core, pl.ds(sid * rpt, rpt)])


@functools.partial(jax.jit, static_argnums=(4,))
def _sc_gather_scatter(tab, row3, col3, zeros, n_acc):
    mesh = plsc.VectorSubcoreMesh(core_axis_name="c", subcore_axis_name="s")
    return pl.kernel(
        _sc_gs_body,
        out_type=jax.ShapeDtypeStruct((_NC, n_acc, _CH), jnp.float32),
        mesh=mesh,
        scratch_types=[
            pltpu.VMEM((_NCHUNK, _K), jnp.int32),
            pltpu.VMEM((_CB, _K), jnp.int32),
            pltpu.VMEM((_K, _CH), jnp.float32),
            pltpu.VMEM((_K, _CH), jnp.float32),
            pltpu.VMEM_SHARED((n_acc, _CH), jnp.float32),
            pltpu.SemaphoreType.DMA,
            pltpu.SemaphoreType.DMA,
            pltpu.SemaphoreType.DMA,
            pltpu.SemaphoreType.DMA,
        ],
    )(tab, row3, col3, zeros)


def _tc_xw_body(state_ref, w_ref, xw_ref):
    xw_ref[...] = jnp.dot(state_ref[...], w_ref[...],
                          preferred_element_type=jnp.float32)


@jax.jit
def _tc_xw(state, conv_W):
    grid = 25
    rb = _N // grid
    return pl.pallas_call(
        _tc_xw_body,
        grid=(grid,),
        in_specs=[
            pl.BlockSpec((rb, _CH), lambda i: (i, 0)),
            pl.BlockSpec((_CH, _CH), lambda i: (0, 0)),
        ],
        out_specs=pl.BlockSpec((rb, _CH), lambda i: (i, 0)),
        out_shape=jax.ShapeDtypeStruct((_N, _CH), jnp.float32),
    )(state, conv_W)


def _tc_xs_body(xw_ref, h0_ref, h1_ref, xs_ref, dis_ref):
    deg = h0_ref[...] + h1_ref[...] + 1.0
    dis = lax.rsqrt(deg)
    xs_ref[...] = dis * xw_ref[...]
    dis_ref[...] = dis


@jax.jit
def _tc_xs(xw, h0, h1):
    grid = 25
    rb = _N // grid
    return pl.pallas_call(
        _tc_xs_body,
        grid=(grid,),
        in_specs=[
            pl.BlockSpec((rb, _CH), lambda i: (i, 0)),
            pl.BlockSpec((rb, 1), lambda i: (i, 0)),
            pl.BlockSpec((rb, 1), lambda i: (i, 0)),
        ],
        out_specs=[
            pl.BlockSpec((rb, _CH), lambda i: (i, 0)),
            pl.BlockSpec((rb, 1), lambda i: (i, 0)),
        ],
        out_shape=[
            jax.ShapeDtypeStruct((_N, _CH), jnp.float32),
            jax.ShapeDtypeStruct((_N, 1), jnp.float32),
        ],
    )(xw, h0, h1)


def _tc_final_body(a0_ref, a1_ref, xs_ref, dis_ref, state_ref, cb_ref, s_ref,
                   w1_ref, b1_ref, w2_ref, b2_ref, w3_ref, b3_ref, out_ref):
    conv = dis_ref[...] * (a0_ref[...] + a1_ref[...] + xs_ref[...]) + cb_ref[...]
    h = jnp.maximum(conv, 0.0) + state_ref[...]
    g = jnp.dot(s_ref[...], h, preferred_element_type=jnp.float32)
    z = jnp.maximum(jnp.dot(g, w1_ref[...], preferred_element_type=jnp.float32)
                    + b1_ref[...], 0.0)
    z = jnp.maximum(jnp.dot(z, w2_ref[...], preferred_element_type=jnp.float32)
                    + b2_ref[...], 0.0)
    y = jnp.dot(z, w3_ref[...], preferred_element_type=jnp.float32) + b3_ref[...]
    out_ref[...] = jnp.broadcast_to(y[None], out_ref.shape)


@jax.jit
def _tc_final(a0, a1, xs, dis, state, conv_b, sel,
              lin1_W, lin1_b, lin2_W, lin2_b, lin3_W, lin3_b):
    grid = 25
    rb = _N // grid
    gb = rb // 16
    full = lambda shape: pl.BlockSpec(shape, lambda i: tuple(0 for _ in shape))
    return pl.pallas_call(
        _tc_final_body,
        grid=(grid,),
        in_specs=[
            pl.BlockSpec((rb, _CH), lambda i: (i, 0)),
            pl.BlockSpec((rb, _CH), lambda i: (i, 0)),
            pl.BlockSpec((rb, _CH), lambda i: (i, 0)),
            pl.BlockSpec((rb, 1), lambda i: (i, 0)),
            pl.BlockSpec((rb, _CH), lambda i: (i, 0)),
            full((1, _CH)),
            full((gb, rb)),
            full((_CH, 64)),
            full((1, 64)),
            full((64, 64)),
            full((1, 64)),
            full((64, 1)),
            full((1, 1)),
        ],
        out_specs=pl.BlockSpec((1, gb, _CH), lambda i: (i, 0, 0)),
        out_shape=jax.ShapeDtypeStruct((grid, gb, _CH), jnp.float32),
    )(a0, a1, xs, dis, state, conv_b, sel,
      lin1_W, lin1_b, lin2_W, lin2_b, lin3_W, lin3_b)


def kernel(state, edge_index, conv_W, conv_b, lin1_W, lin1_b, lin2_W, lin2_b,
           lin3_W, lin3_b):
    npad = _NW * _EPWP - _E
    arp = jnp.arange(npad, dtype=jnp.int32)
    row = jnp.concatenate([edge_index[0].astype(jnp.int32), (arp * 7) % _N])
    col = edge_index[1].astype(jnp.int32)
    row32 = row.reshape(_NW, _NCHUNK, _K)
    col32 = jnp.concatenate([col, _N + (arp % (_NP - _N))]).reshape(
        _NW, _NCHUNK, _K)
    zeros = jnp.zeros((_NP, _CH), jnp.float32)

    ar = jnp.arange(_E, dtype=jnp.int32)
    hrow = jnp.concatenate([(col & 15) * _REP + (ar % _REP),
                            arp % (16 * _REP)]).reshape(_NW, _NCHUNK, _K)
    hcol = jnp.concatenate([col >> 4,
                            625 + (arp % (_NH - 625))]).reshape(
        _NW, _NCHUNK, _K)
    m = jnp.arange(16 * _REP, dtype=jnp.int32)
    onehot = jnp.zeros((16 * _REP, _CH), jnp.float32).at[m, 8 * (m // _REP)].set(1.0)
    zeros_h = jnp.zeros((_NH, _CH), jnp.float32)
    hist = _sc_gather_scatter(onehot, hrow, hcol, zeros_h, _NH)
    h0 = hist[0].reshape(_NH, 16, 8)[:, :, 0].reshape(-1, 1)[:_N]
    h1 = hist[1].reshape(_NH, 16, 8)[:, :, 0].reshape(-1, 1)[:_N]

    xw = _tc_xw(state, conv_W)
    xs, dis = _tc_xs(xw, h0, h1)
    acc = _sc_gather_scatter(xs, row32, col32, zeros, _NP)

    sel = (jnp.arange(25)[:, None] == (jnp.arange(400) // 16)[None, :]
           ).astype(jnp.float32)
    out3d = _tc_final(acc[0, :_N], acc[1, :_N], xs, dis, state,
                      conv_b.reshape(1, _CH), sel, lin1_W,
                      lin1_b.reshape(1, 64), lin2_W, lin2_b.reshape(1, 64),
                      lin3_W, lin3_b.reshape(1, 1))
    return out3d.reshape(_N // 16, _CH)[:, 0]
